# baseline (speedup 1.0000x reference)
import jax
import jax.numpy as jnp
from jax.experimental import pallas as pl
from jax.experimental.pallas import tpu as pltpu

_N = 10000
_D = 128
_TM = 80
_HW_CHUNK = 1000


def _body(h_ref, a_ref, w_ref, b_ref, out_ref, hw_ref):
    m = pl.program_id(0)

    @pl.when(m == 0)
    def _init_hw():
        for i in range(_N // _HW_CHUNK):
            sl = slice(i * _HW_CHUNK, (i + 1) * _HW_CHUNK)
            hw_ref[sl, :] = jnp.dot(
                h_ref[sl, :], w_ref[...], preferred_element_type=jnp.float32
            )

    out_ref[...] = a_ref[:, : _D] + b_ref[...]


def kernel(H, A, W, b):
    b2 = b.reshape(1, _D)
    return pl.pallas_call(
        _body,
        grid=(_N // _TM,),
        in_specs=[
            pl.BlockSpec((_N, _D), lambda m: (0, 0)),
            pl.BlockSpec((_TM, _N), lambda m: (m, 0)),
            pl.BlockSpec((_D, _D), lambda m: (0, 0)),
            pl.BlockSpec((1, _D), lambda m: (0, 0)),
        ],
        out_specs=pl.BlockSpec((_TM, _D), lambda m: (m, 0)),
        out_shape=jax.ShapeDtypeStruct((_N, _D), jnp.float32),
        scratch_shapes=[pltpu.VMEM((_N, _D), jnp.float32)],
        compiler_params=pltpu.CompilerParams(
            dimension_semantics=("arbitrary",),
        ),
    )(H, A, W, b2)


# speedup vs baseline: 1.0417x; 1.0417x over previous
import jax
import jax.numpy as jnp
from jax import lax
from jax.experimental import pallas as pl
from jax.experimental.pallas import tpu as pltpu

_N = 10000
_D = 128
_TM = 200
_NB = 5
_S = _N // _TM
_G = _S // _NB
_HW_CHUNK = 1000


def _body(h_ref, a_ref, w_ref, b_ref, out_ref, hw_ref, ring, obuf, isem, osem):
    for j in range(_NB):
        pltpu.make_async_copy(
            a_ref.at[pl.ds(j * _TM, _TM), :], ring.at[j], isem.at[j]
        ).start()

    for i in range(_N // _HW_CHUNK):
        sl = slice(i * _HW_CHUNK, (i + 1) * _HW_CHUNK)
        hw_ref[sl, :] = jnp.dot(
            h_ref[sl, :], w_ref[...], preferred_element_type=jnp.float32
        ).astype(jnp.bfloat16)

    def group(g, carry):
        base = g * _NB * _TM
        for j in range(_NB):
            row = base + j * _TM
            pltpu.make_async_copy(
                a_ref.at[pl.ds(row, _TM), :], ring.at[j], isem.at[j]
            ).wait()

            @pl.when(g > 0)
            def _wait_out():
                pltpu.make_async_copy(
                    obuf.at[j],
                    out_ref.at[pl.ds(row - _NB * _TM, _TM), :],
                    osem.at[j],
                ).wait()

            obuf[j] = (
                jnp.dot(
                    ring[j].astype(jnp.bfloat16),
                    hw_ref[...],
                    preferred_element_type=jnp.float32,
                )
                + b_ref[...]
            )
            pltpu.make_async_copy(
                obuf.at[j], out_ref.at[pl.ds(row, _TM), :], osem.at[j]
            ).start()

            @pl.when(g < _G - 1)
            def _next_fetch():
                pltpu.make_async_copy(
                    a_ref.at[pl.ds(row + _NB * _TM, _TM), :],
                    ring.at[j],
                    isem.at[j],
                ).start()

        return carry

    lax.fori_loop(0, _G, group, 0)

    for j in range(_NB):
        row = (_G - 1) * _NB * _TM + j * _TM
        pltpu.make_async_copy(
            obuf.at[j], out_ref.at[pl.ds(row, _TM), :], osem.at[j]
        ).wait()


def kernel(H, A, W, b):
    b2 = b.reshape(1, _D)
    return pl.pallas_call(
        _body,
        in_specs=[
            pl.BlockSpec(memory_space=pltpu.VMEM),
            pl.BlockSpec(memory_space=pl.ANY),
            pl.BlockSpec(memory_space=pltpu.VMEM),
            pl.BlockSpec(memory_space=pltpu.VMEM),
        ],
        out_specs=pl.BlockSpec(memory_space=pl.ANY),
        out_shape=jax.ShapeDtypeStruct((_N, _D), jnp.float32),
        scratch_shapes=[
            pltpu.VMEM((_N, _D), jnp.bfloat16),
            pltpu.VMEM((_NB, _TM, _N), jnp.float32),
            pltpu.VMEM((_NB, _TM, _D), jnp.float32),
            pltpu.SemaphoreType.DMA((_NB,)),
            pltpu.SemaphoreType.DMA((_NB,)),
        ],
    )(H, A, W, b2)
